# Initial kernel scaffold; baseline (speedup 1.0000x reference)
import jax
import jax.numpy as jnp
from jax.experimental import pallas as pl


def kernel(x, edge_index, W1, b1, W2, b2):
    raise NotImplementedError("write your pallas kernel here")


# speedup vs baseline: 14.9275x; 14.9275x over previous
import functools

import jax
import jax.numpy as jnp
from jax import lax
from jax.experimental import pallas as pl
from jax.experimental.pallas import tpu as pltpu
from jax.experimental.pallas import tpu_sc as plsc

N = 10000
E = 320000
D_IN = 128
D_HID = 128
D_OUT = 64

NC = 2
NS = 16
NW = NC * NS
K = 128
J = 79
E_PAD = NW * J * K
N_PAD = 10240
STRIPE = N_PAD // NS
DUMMY = N

_mesh = plsc.VectorSubcoreMesh(
    core_axis_name="c", subcore_axis_name="s", num_cores=NC, num_subcores=NS)


@functools.partial(
    pl.kernel,
    out_type=jax.ShapeDtypeStruct((NW, N_PAD), jnp.float32),
    mesh=_mesh,
    compiler_params=pltpu.CompilerParams(needs_layout_passes=False),
    scratch_types=[
        pltpu.VMEM((J, K), jnp.int32),
        pltpu.VMEM((N_PAD,), jnp.float32),
    ],
)
def _deg_kernel(dst_hbm, out_hbm, idx_v, hist_v):
    c = lax.axis_index("c")
    s = lax.axis_index("s")
    w = c * NS + s
    pltpu.sync_copy(dst_hbm.at[w], idx_v)

    zeros16 = jnp.zeros((16,), jnp.float32)

    def zero_body(i, carry):
        hist_v[pl.ds(i * 16, 16)] = zeros16
        return carry

    lax.fori_loop(0, N_PAD // 16, zero_body, 0)

    ones16 = jnp.ones((16,), jnp.float32)

    def hist_body(i, carry):
        j = i // (K // 16)
        k = i % (K // 16)
        idx = idx_v[j, pl.ds(k * 16, 16)]
        plsc.addupdate_scatter(hist_v, [idx], ones16)
        return carry

    lax.fori_loop(0, (J * K) // 16, hist_body, 0)
    pltpu.sync_copy(hist_v, out_hbm.at[w])


def _make_scatter(D):
    @functools.partial(
        pl.kernel,
        out_type=jax.ShapeDtypeStruct((NC, N_PAD, D), jnp.float32),
        mesh=_mesh,
        compiler_params=pltpu.CompilerParams(use_tc_tiling_on_sc=False),
        scratch_types=[
            pltpu.VMEM((J, K), jnp.int32),
            pltpu.VMEM((J, K), jnp.int32),
            pltpu.VMEM((K, D), jnp.float32),
            pltpu.VMEM_SHARED((N_PAD, D), jnp.float32),
            pltpu.SemaphoreType.DMA,
        ],
    )
    def _scatter_kernel(g_hbm, src_hbm, dst_hbm, out_hbm,
                        src_v, dst_v, rows_v, acc_sh, sem):
        c = lax.axis_index("c")
        s = lax.axis_index("s")
        w = c * NS + s
        pltpu.sync_copy(src_hbm.at[w], src_v)
        pltpu.sync_copy(dst_hbm.at[w], dst_v)

        zeros16 = jnp.zeros((16,), jnp.float32)

        def zrow(i, carry):
            r = i // (D // 16)
            l = i % (D // 16)
            rows_v[r, pl.ds(l * 16, 16)] = zeros16
            return carry

        lax.fori_loop(0, (K * D) // 16, zrow, 0)
        for p in range(STRIPE // K):
            pltpu.sync_copy(rows_v, acc_sh.at[pl.ds(s * STRIPE + p * K, K)])
        plsc.subcore_barrier()

        def body(j, carry):
            pltpu.async_copy(g_hbm.at[src_v.at[j]], rows_v, sem).wait()
            pltpu.sync_copy(rows_v, acc_sh.at[dst_v.at[j]], add=True)
            return carry

        lax.fori_loop(0, J, body, 0)
        plsc.subcore_barrier()
        pltpu.sync_copy(acc_sh.at[pl.ds(s * STRIPE, STRIPE)],
                        out_hbm.at[c, pl.ds(s * STRIPE, STRIPE)])

    return _scatter_kernel


_scatter128 = _make_scatter(D_HID)
_scatter64 = _make_scatter(D_OUT)


_R = 1024


def _dinv_of(hist_blk):
    deg = jnp.sum(hist_blk, axis=1, keepdims=True) + 1.0
    return lax.rsqrt(deg)


def _tc1_body(hist_ref, x_ref, w1_ref, g1_ref):
    dinv = _dinv_of(hist_ref[...])
    g1_ref[...] = jnp.dot(x_ref[...], w1_ref[...],
                          preferred_element_type=jnp.float32) * dinv


def _tc2_body(hist_ref, acc_ref, g1_ref, b1_ref, w2_ref, g2_ref):
    dinv = _dinv_of(hist_ref[...])
    a = acc_ref[0] + acc_ref[1] + g1_ref[...]
    h = jnp.maximum(a * dinv + b1_ref[...], 0.0)
    g2_ref[...] = jnp.dot(h, w2_ref[...],
                          preferred_element_type=jnp.float32) * dinv


def _tc3_body(hist_ref, acc_ref, g2_ref, b2_ref, out_ref):
    dinv = _dinv_of(hist_ref[...])
    a = acc_ref[0] + acc_ref[1] + g2_ref[...]
    out_ref[...] = jnp.maximum(a * dinv + b2_ref[...], 0.0)


def _row_spec(d):
    return pl.BlockSpec((_R, d), lambda i: (i, 0))


def _acc_spec(d):
    return pl.BlockSpec((NC, _R, d), lambda i: (0, i, 0))


def _full_spec(r, d):
    return pl.BlockSpec((r, d), lambda i: (0, 0))


_GRID = N_PAD // _R

_tc1 = pl.pallas_call(
    _tc1_body,
    grid=(_GRID,),
    in_specs=[_row_spec(NW), _row_spec(D_IN), _full_spec(D_IN, D_HID)],
    out_specs=_row_spec(D_HID),
    out_shape=jax.ShapeDtypeStruct((N_PAD, D_HID), jnp.float32),
)

_tc2 = pl.pallas_call(
    _tc2_body,
    grid=(_GRID,),
    in_specs=[_row_spec(NW), _acc_spec(D_HID), _row_spec(D_HID),
              _full_spec(1, D_HID), _full_spec(D_HID, D_OUT)],
    out_specs=_row_spec(D_OUT),
    out_shape=jax.ShapeDtypeStruct((N_PAD, D_OUT), jnp.float32),
)

_tc3 = pl.pallas_call(
    _tc3_body,
    grid=(_GRID,),
    in_specs=[_row_spec(NW), _acc_spec(D_OUT), _row_spec(D_OUT),
              _full_spec(1, D_OUT)],
    out_specs=_row_spec(D_OUT),
    out_shape=jax.ShapeDtypeStruct((N_PAD, D_OUT), jnp.float32),
)


def kernel(x, edge_index, W1, b1, W2, b2):
    x_pad = jnp.zeros((N_PAD, D_IN), x.dtype).at[:N].set(x)
    pad = jnp.full((E_PAD - E,), DUMMY, jnp.int32)
    src = jnp.concatenate([edge_index[0], pad]).reshape(NW, J, K)
    dst = jnp.concatenate([edge_index[1], pad]).reshape(NW, J, K)

    hist = _deg_kernel(dst)
    hist_t = hist.T

    g1 = _tc1(hist_t, x_pad, W1)
    acc1 = _scatter128(g1, src, dst)
    g2 = _tc2(hist_t, acc1, g1, b1.reshape(1, D_HID), W2)
    acc2 = _scatter64(g2, src, dst)
    out = _tc3(hist_t, acc2, g2, b2.reshape(1, D_OUT))
    return out[:N]


# speedup vs baseline: 17.7120x; 1.1865x over previous
import functools

import jax
import jax.numpy as jnp
from jax import lax
from jax.experimental import pallas as pl
from jax.experimental.pallas import tpu as pltpu
from jax.experimental.pallas import tpu_sc as plsc

N = 10000
E = 320000
D_IN = 128
D_HID = 128
D_OUT = 64

NC = 2
NS = 16
NW = NC * NS
K = 128
J = 79
E_PAD = NW * J * K
N_PAD = 10240
STRIPE = N_PAD // NS
DUMMY = N

_mesh = plsc.VectorSubcoreMesh(
    core_axis_name="c", subcore_axis_name="s", num_cores=NC, num_subcores=NS)


@functools.partial(
    pl.kernel,
    out_type=jax.ShapeDtypeStruct((NW, N_PAD), jnp.float32),
    mesh=_mesh,
    compiler_params=pltpu.CompilerParams(needs_layout_passes=False),
    scratch_types=[
        pltpu.VMEM((J, K), jnp.int32),
        pltpu.VMEM((N_PAD,), jnp.float32),
    ],
)
def _deg_kernel(dst_hbm, out_hbm, idx_v, hist_v):
    c = lax.axis_index("c")
    s = lax.axis_index("s")
    w = c * NS + s
    pltpu.sync_copy(dst_hbm.at[w], idx_v)

    zeros16 = jnp.zeros((16,), jnp.float32)

    def zero_body(i, carry):
        hist_v[pl.ds(i * 16, 16)] = zeros16
        return carry

    lax.fori_loop(0, N_PAD // 16, zero_body, 0)

    ones16 = jnp.ones((16,), jnp.float32)

    def hist_body(i, carry):
        j = i // (K // 16)
        k = i % (K // 16)
        idx = idx_v[j, pl.ds(k * 16, 16)]
        plsc.addupdate_scatter(hist_v, [idx], ones16)
        return carry

    lax.fori_loop(0, (J * K) // 16, hist_body, 0)
    pltpu.sync_copy(hist_v, out_hbm.at[w])


def _make_scatter(D):
    @functools.partial(
        pl.kernel,
        out_type=jax.ShapeDtypeStruct((NC, N_PAD, D), jnp.float32),
        mesh=_mesh,
        compiler_params=pltpu.CompilerParams(use_tc_tiling_on_sc=False),
        scratch_types=[
            pltpu.VMEM((J, K), jnp.int32),
            pltpu.VMEM((J, K), jnp.int32),
            pltpu.VMEM((K, D), jnp.float32),
            pltpu.VMEM((K, D), jnp.float32),
            pltpu.VMEM_SHARED((N_PAD, D), jnp.float32),
            pltpu.SemaphoreType.DMA,
            pltpu.SemaphoreType.DMA,
        ],
    )
    def _scatter_kernel(g_hbm, src_hbm, dst_hbm, out_hbm,
                        src_v, dst_v, rows0, rows1, acc_sh, sem0, sem1):
        c = lax.axis_index("c")
        s = lax.axis_index("s")
        w = c * NS + s
        pltpu.sync_copy(src_hbm.at[w], src_v)
        pltpu.sync_copy(dst_hbm.at[w], dst_v)

        pltpu.async_copy(g_hbm.at[src_v.at[0]], rows0, sem0)

        zeros16 = jnp.zeros((16,), jnp.float32)

        def zrow(i, carry):
            r = i // (D // 16)
            l = i % (D // 16)
            rows1[r, pl.ds(l * 16, 16)] = zeros16
            return carry

        lax.fori_loop(0, (K * D) // 16, zrow, 0)
        for p in range(STRIPE // K):
            pltpu.sync_copy(rows1, acc_sh.at[pl.ds(s * STRIPE + p * K, K)])
        plsc.subcore_barrier()

        def body(jp, carry):
            j0 = 2 * jp
            pltpu.async_copy(g_hbm.at[src_v.at[j0 + 1]], rows1, sem1)
            pltpu.make_async_copy(g_hbm.at[src_v.at[j0]], rows0, sem0).wait()
            pltpu.sync_copy(rows0, acc_sh.at[dst_v.at[j0]], add=True)
            pltpu.async_copy(g_hbm.at[src_v.at[j0 + 2]], rows0, sem0)
            pltpu.make_async_copy(g_hbm.at[src_v.at[j0 + 1]], rows1, sem1).wait()
            pltpu.sync_copy(rows1, acc_sh.at[dst_v.at[j0 + 1]], add=True)
            return carry

        lax.fori_loop(0, (J - 1) // 2, body, 0)
        pltpu.make_async_copy(g_hbm.at[src_v.at[J - 1]], rows0, sem0).wait()
        pltpu.sync_copy(rows0, acc_sh.at[dst_v.at[J - 1]], add=True)
        plsc.subcore_barrier()
        pltpu.sync_copy(acc_sh.at[pl.ds(s * STRIPE, STRIPE)],
                        out_hbm.at[c, pl.ds(s * STRIPE, STRIPE)])

    return _scatter_kernel


_scatter64 = _make_scatter(D_OUT)


_R = 1024


def _dinv_of(hist_blk):
    deg = jnp.sum(hist_blk, axis=1, keepdims=True) + 1.0
    return lax.rsqrt(deg)


def _tc1_body(hist_ref, x_ref, w1_ref, g1a_ref, g1b_ref):
    dinv = _dinv_of(hist_ref[...])
    g1 = jnp.dot(x_ref[...], w1_ref[...],
                 preferred_element_type=jnp.float32) * dinv
    g1a_ref[...] = g1[:, :D_OUT]
    g1b_ref[...] = g1[:, D_OUT:]


def _tc2_body(hist_ref, acca_ref, accb_ref, g1a_ref, g1b_ref,
              b1_ref, w2a_ref, w2b_ref, g2_ref):
    dinv = _dinv_of(hist_ref[...])
    ha = jnp.maximum((acca_ref[0] + acca_ref[1] + g1a_ref[...]) * dinv
                     + b1_ref[:, :D_OUT], 0.0)
    hb = jnp.maximum((accb_ref[0] + accb_ref[1] + g1b_ref[...]) * dinv
                     + b1_ref[:, D_OUT:], 0.0)
    g2 = (jnp.dot(ha, w2a_ref[...], preferred_element_type=jnp.float32)
          + jnp.dot(hb, w2b_ref[...], preferred_element_type=jnp.float32))
    g2_ref[...] = g2 * dinv


def _tc3_body(hist_ref, acc_ref, g2_ref, b2_ref, out_ref):
    dinv = _dinv_of(hist_ref[...])
    a = acc_ref[0] + acc_ref[1] + g2_ref[...]
    out_ref[...] = jnp.maximum(a * dinv + b2_ref[...], 0.0)


def _row_spec(d):
    return pl.BlockSpec((_R, d), lambda i: (i, 0))


def _acc_spec(d):
    return pl.BlockSpec((NC, _R, d), lambda i: (0, i, 0))


def _full_spec(r, d):
    return pl.BlockSpec((r, d), lambda i: (0, 0))


_GRID = N_PAD // _R

_tc1 = pl.pallas_call(
    _tc1_body,
    grid=(_GRID,),
    in_specs=[_row_spec(NW), _row_spec(D_IN), _full_spec(D_IN, D_HID)],
    out_specs=[_row_spec(D_OUT), _row_spec(D_OUT)],
    out_shape=[jax.ShapeDtypeStruct((N_PAD, D_OUT), jnp.float32),
               jax.ShapeDtypeStruct((N_PAD, D_OUT), jnp.float32)],
)

_tc2 = pl.pallas_call(
    _tc2_body,
    grid=(_GRID,),
    in_specs=[_row_spec(NW), _acc_spec(D_OUT), _acc_spec(D_OUT),
              _row_spec(D_OUT), _row_spec(D_OUT),
              _full_spec(1, D_HID), _full_spec(D_OUT, D_OUT),
              _full_spec(D_OUT, D_OUT)],
    out_specs=_row_spec(D_OUT),
    out_shape=jax.ShapeDtypeStruct((N_PAD, D_OUT), jnp.float32),
)

_tc3 = pl.pallas_call(
    _tc3_body,
    grid=(_GRID,),
    in_specs=[_row_spec(NW), _acc_spec(D_OUT), _row_spec(D_OUT),
              _full_spec(1, D_OUT)],
    out_specs=_row_spec(D_OUT),
    out_shape=jax.ShapeDtypeStruct((N_PAD, D_OUT), jnp.float32),
)


def kernel(x, edge_index, W1, b1, W2, b2):
    x_pad = jnp.zeros((N_PAD, D_IN), x.dtype).at[:N].set(x)
    pad = jnp.full((E_PAD - E,), DUMMY, jnp.int32)
    src = jnp.concatenate([edge_index[0], pad]).reshape(NW, J, K)
    dst = jnp.concatenate([edge_index[1], pad]).reshape(NW, J, K)

    hist = _deg_kernel(dst)
    hist_t = hist.T

    g1a, g1b = _tc1(hist_t, x_pad, W1)
    acc1a = _scatter64(g1a, src, dst)
    acc1b = _scatter64(g1b, src, dst)
    g2 = _tc2(hist_t, acc1a, acc1b, g1a, g1b, b1.reshape(1, D_HID),
              W2[:D_OUT], W2[D_OUT:])
    acc2 = _scatter64(g2, src, dst)
    out = _tc3(hist_t, acc2, g2, b2.reshape(1, D_OUT))
    return out[:N]
